# baseline (speedup 1.0000x reference)
import functools

import jax
import jax.numpy as jnp
from jax import lax
from jax.experimental import pallas as pl
from jax.experimental.pallas import tpu as pltpu
from jax.experimental.pallas import tpu_sc as plsc

N_NODES = 10000
N_EDGES = 320000
D = 128
CHUNK = 128
N_CHUNKS = N_EDGES // CHUNK
NC = 2
NS = 16
NW = NC * NS
N_PAD = 10240
ROWS_PER_TILE = N_PAD // NS
MAX_CH = 79
RING = 3
BLOCKS = (MAX_CH + RING - 1) // RING

_sc_mesh = plsc.VectorSubcoreMesh(core_axis_name="c", subcore_axis_name="s",
                                  num_cores=NC, num_subcores=NS)


def _deg_body(ei_hbm, out_hbm, idx_v, hist_v, red_v, outloc_v, stage_sh, sem):
    c = lax.axis_index("c")
    s = lax.axis_index("s")
    wid = s * NC + c
    zero16 = jnp.zeros((16,), jnp.float32)
    ones16 = jnp.full((16,), 1.0, jnp.float32)

    eoff = pl.multiple_of(N_EDGES + wid * (N_EDGES // NW), 8)
    idx_dma = pltpu.async_copy(ei_hbm.at[pl.ds(eoff, N_EDGES // NW)], idx_v,
                               sem)

    def zbody(i, carry):
        hist_v[pl.ds(pl.multiple_of(i * 16, 16), 16)] = zero16
        return carry

    lax.fori_loop(0, N_PAD // 16, zbody, 0)
    idx_dma.wait()

    def body(m, carry):
        idx16 = idx_v[pl.ds(pl.multiple_of(m * 16, 16), 16)]
        plsc.addupdate_scatter(hist_v, [idx16], ones16)
        return carry

    lax.fori_loop(0, N_EDGES // NW // 16, body, 0)

    pltpu.sync_copy(hist_v, stage_sh.at[s])
    plsc.subcore_barrier()
    rbase = pl.multiple_of(s * ROWS_PER_TILE, 8)
    pltpu.sync_copy(stage_sh.at[:, pl.ds(rbase, ROWS_PER_TILE)], red_v)
    lane0 = jnp.zeros((16,), jnp.int32)

    def rbody(m, carry):
        o = pl.multiple_of(m * 16, 16)
        acc = red_v[0, pl.ds(o, 16)]
        for t in range(1, NS):
            acc = acc + red_v[t, pl.ds(o, 16)]
        rows = m * 16 + lax.iota(jnp.int32, 16)
        plsc.store_scatter(outloc_v, [rows, lane0], acc)
        return carry

    lax.fori_loop(0, ROWS_PER_TILE // 16, rbody, 0)
    pltpu.sync_copy(outloc_v, out_hbm.at[c, pl.ds(rbase, ROWS_PER_TILE)])


_deg_call = functools.partial(
    pl.kernel,
    out_type=jax.ShapeDtypeStruct((NC, N_PAD, D), jnp.float32),
    mesh=_sc_mesh,
    scratch_types=[
        pltpu.VMEM((N_EDGES // NW,), jnp.int32),
        pltpu.VMEM((N_PAD,), jnp.float32),
        pltpu.VMEM((NS, ROWS_PER_TILE), jnp.float32),
        pltpu.VMEM((ROWS_PER_TILE, D), jnp.float32),
        pltpu.VMEM_SHARED((NS, N_PAD), jnp.float32),
        pltpu.SemaphoreType.DMA,
    ],
    compiler_params=pltpu.CompilerParams(needs_layout_passes=False,
                                         use_tc_tiling_on_sc=False),
)(_deg_body)


MSG_RPT = 632
MCH = 80
MCHUNKS = N_EDGES // MCH
MCOUNT = MCHUNKS // NW
MRING = 4
MBLOCKS = (MCOUNT + MRING - 1) // MRING


def _msg_body(ei_hbm, g_hbm, out_hbm, *refs):
    ridx = refs[0:MRING]
    cidx = refs[MRING:2 * MRING]
    bufs = refs[2 * MRING:3 * MRING]
    acc_sh = refs[3 * MRING]
    gsem = refs[3 * MRING + 1]
    ssem = refs[3 * MRING + 2]
    isem = refs[3 * MRING + 3]

    c = lax.axis_index("c")
    s = lax.axis_index("s")
    wid = s * NC + c
    base = pl.multiple_of(s * MSG_RPT, 8)
    count = MCOUNT

    def fill(j, k):
        off = pl.multiple_of((wid + j * NW) * MCH, 8)
        pltpu.async_copy(ei_hbm.at[pl.ds(off, MCH)], ridx[k], isem.at[k])
        pltpu.async_copy(ei_hbm.at[pl.ds(N_EDGES + off, MCH)], cidx[k],
                         isem.at[k])

    def iwait(k):
        pltpu.make_async_copy(ei_hbm.at[pl.ds(0, MCH)], ridx[k],
                              isem.at[k]).wait()
        pltpu.make_async_copy(ei_hbm.at[pl.ds(0, MCH)], cidx[k],
                              isem.at[k]).wait()

    def gather(k):
        pltpu.async_copy(g_hbm.at[ridx[k]], bufs[k], gsem.at[k])

    def gwait(k):
        pltpu.make_async_copy(g_hbm.at[pl.ds(0, MCH)], bufs[k],
                              gsem.at[k]).wait()

    def scat(k):
        pltpu.async_copy(bufs[k], acc_sh.at[cidx[k]], ssem.at[k], add=True)

    def swait(k):
        pltpu.make_async_copy(bufs[k], acc_sh.at[cidx[k]], ssem.at[k]).wait()

    for k in range(MRING):
        fill(k, k)

    zero16 = jnp.zeros((16,), jnp.float32)

    def zrow(r, carry):
        for k8 in range(D // 16):
            bufs[0][r, pl.ds(k8 * 16, 16)] = zero16
        return carry

    lax.fori_loop(0, MCH, zrow, 0)
    for t in range(6):
        pltpu.sync_copy(bufs[0], acc_sh.at[pl.ds(base + t * MCH, MCH)])

    @pl.when(s < NS - 1)
    def _():
        pltpu.sync_copy(bufs[0], acc_sh.at[pl.ds(base + 6 * MCH, MCH)])
        pltpu.sync_copy(bufs[0].at[pl.ds(0, MSG_RPT - 7 * MCH)],
                        acc_sh.at[pl.ds(base + 7 * MCH, MSG_RPT - 7 * MCH)])

    @pl.when(s == NS - 1)
    def _():
        pltpu.sync_copy(bufs[0].at[pl.ds(0, N_NODES - 15 * MSG_RPT - 6 * MCH)],
                        acc_sh.at[pl.ds(base + 6 * MCH,
                                        N_NODES - 15 * MSG_RPT - 6 * MCH)])

    plsc.subcore_barrier()

    for k in range(MRING):
        iwait(k)
        gather(k)

    def block(b, carry):
        jb = b * MRING
        for k in range(MRING):
            def consume(k=k):
                gwait(k)
                scat(k)
            pl.when(jb + k < count)(consume)
        for k in range(MRING):
            jn = jb + MRING + k

            def refill(jn=jn, k=k):
                swait(k)
                fill(jn, k)
                iwait(k)
                gather(k)
            pl.when(jn < count)(refill)
        return carry

    lax.fori_loop(0, MBLOCKS, block, 0)
    for k in range(MRING):
        swait(k)
    plsc.subcore_barrier()

    @pl.when(s < NS - 1)
    def _():
        sl = pl.ds(base, MSG_RPT)
        pltpu.sync_copy(acc_sh.at[sl], out_hbm.at[c, sl])

    @pl.when(s == NS - 1)
    def _():
        sl = pl.ds(base, N_NODES - 15 * MSG_RPT)
        pltpu.sync_copy(acc_sh.at[sl], out_hbm.at[c, sl])


_msg_call = functools.partial(
    pl.kernel,
    out_type=jax.ShapeDtypeStruct((NC, N_NODES, D), jnp.float32),
    mesh=_sc_mesh,
    scratch_types=(
        [pltpu.VMEM((MCH,), jnp.int32)] * (2 * MRING)
        + [pltpu.VMEM((MCH, D), jnp.float32)] * MRING
        + [
            pltpu.VMEM_SHARED((N_NODES, D), jnp.float32),
            pltpu.SemaphoreType.DMA((MRING,)),
            pltpu.SemaphoreType.DMA((MRING,)),
            pltpu.SemaphoreType.DMA((MRING,)),
        ]
    ),
)(_msg_body)


TC_BLK = 2000


def _g_body(x_ref, w_ref, b_ref, degp_ref, g_ref):
    h = lax.dot_general(x_ref[...], w_ref[...],
                        (((1,), (1,)), ((), ())),
                        preferred_element_type=jnp.float32) + b_ref[...]
    d = degp_ref[0, :, :1] + degp_ref[1, :, :1]
    dis = jnp.where(d > 0, lax.rsqrt(d), 0.0)
    g_ref[...] = h * dis


def _g_call(x, W, b2d, degp):
    return pl.pallas_call(
        _g_body,
        grid=(N_NODES // TC_BLK,),
        in_specs=[
            pl.BlockSpec((TC_BLK, D), lambda i: (i, 0)),
            pl.BlockSpec((D, D), lambda i: (0, 0)),
            pl.BlockSpec((1, D), lambda i: (0, 0)),
            pl.BlockSpec((NC, TC_BLK, D), lambda i: (0, i, 0)),
        ],
        out_specs=pl.BlockSpec((TC_BLK, D), lambda i: (i, 0)),
        out_shape=jax.ShapeDtypeStruct((N_NODES, D), jnp.float32),
    )(x, W, b2d, degp)


def _out_body(p_ref, degp_ref, o_ref):
    d = degp_ref[0, :, :1] + degp_ref[1, :, :1]
    dis = jnp.where(d > 0, lax.rsqrt(d), 0.0)
    o_ref[...] = (p_ref[0] + p_ref[1]) * dis


def _out_call(partial, degp):
    return pl.pallas_call(
        _out_body,
        grid=(N_NODES // TC_BLK,),
        in_specs=[
            pl.BlockSpec((NC, TC_BLK, D), lambda i: (0, i, 0)),
            pl.BlockSpec((NC, TC_BLK, D), lambda i: (0, i, 0)),
        ],
        out_specs=pl.BlockSpec((TC_BLK, D), lambda i: (i, 0)),
        out_shape=jax.ShapeDtypeStruct((N_NODES, D), jnp.float32),
    )(partial, degp)


def kernel(x, edge_index, W, b):
    ei1d = edge_index.reshape(2 * N_EDGES)
    degp = _deg_call(ei1d)
    g = _g_call(x, W, b.reshape(1, D), degp)
    partial = _msg_call(ei1d, g)
    return _out_call(partial, degp)


# speedup vs baseline: 1.0132x; 1.0132x over previous
import functools

import jax
import jax.numpy as jnp
from jax import lax
from jax.experimental import pallas as pl
from jax.experimental.pallas import tpu as pltpu
from jax.experimental.pallas import tpu_sc as plsc

N_NODES = 10000
N_EDGES = 320000
D = 128
CHUNK = 128
N_CHUNKS = N_EDGES // CHUNK
NC = 2
NS = 16
NW = NC * NS
N_PAD = 10240
ROWS_PER_TILE = N_PAD // NS
MAX_CH = 79
RING = 3
BLOCKS = (MAX_CH + RING - 1) // RING

_sc_mesh = plsc.VectorSubcoreMesh(core_axis_name="c", subcore_axis_name="s",
                                  num_cores=NC, num_subcores=NS)


def _deg_body(ei_hbm, out_hbm, idx_v, hist_v, red_v, outloc_v, stage_sh, sem):
    c = lax.axis_index("c")
    s = lax.axis_index("s")
    wid = s * NC + c
    zero16 = jnp.zeros((16,), jnp.float32)
    ones16 = jnp.full((16,), 1.0, jnp.float32)

    eoff = pl.multiple_of(N_EDGES + wid * (N_EDGES // NW), 8)
    idx_dma = pltpu.async_copy(ei_hbm.at[pl.ds(eoff, N_EDGES // NW)], idx_v,
                               sem)

    def zbody(i, carry):
        hist_v[pl.ds(pl.multiple_of(i * 16, 16), 16)] = zero16
        return carry

    lax.fori_loop(0, N_PAD // 16, zbody, 0)
    idx_dma.wait()

    def body(m, carry):
        idx16 = idx_v[pl.ds(pl.multiple_of(m * 16, 16), 16)]
        plsc.addupdate_scatter(hist_v, [idx16], ones16)
        return carry

    lax.fori_loop(0, N_EDGES // NW // 16, body, 0)

    pltpu.sync_copy(hist_v, stage_sh.at[s])
    plsc.subcore_barrier()
    rbase = pl.multiple_of(s * ROWS_PER_TILE, 8)
    pltpu.sync_copy(stage_sh.at[:, pl.ds(rbase, ROWS_PER_TILE)], red_v)
    lane0 = jnp.zeros((16,), jnp.int32)

    def rbody(m, carry):
        o = pl.multiple_of(m * 16, 16)
        acc = red_v[0, pl.ds(o, 16)]
        for t in range(1, NS):
            acc = acc + red_v[t, pl.ds(o, 16)]
        rows = m * 16 + lax.iota(jnp.int32, 16)
        plsc.store_scatter(outloc_v, [rows, lane0], acc)
        return carry

    lax.fori_loop(0, ROWS_PER_TILE // 16, rbody, 0)
    pltpu.sync_copy(outloc_v, out_hbm.at[c, pl.ds(rbase, ROWS_PER_TILE)])


_deg_call = functools.partial(
    pl.kernel,
    out_type=jax.ShapeDtypeStruct((NC, N_PAD, D), jnp.float32),
    mesh=_sc_mesh,
    scratch_types=[
        pltpu.VMEM((N_EDGES // NW,), jnp.int32),
        pltpu.VMEM((N_PAD,), jnp.float32),
        pltpu.VMEM((NS, ROWS_PER_TILE), jnp.float32),
        pltpu.VMEM((ROWS_PER_TILE, D), jnp.float32),
        pltpu.VMEM_SHARED((NS, N_PAD), jnp.float32),
        pltpu.SemaphoreType.DMA,
    ],
    compiler_params=pltpu.CompilerParams(needs_layout_passes=False,
                                         use_tc_tiling_on_sc=False),
)(_deg_body)


MSG_RPT = 632


def _msg_body(ei_hbm, g_hbm, out_hbm, *refs):
    ridx = refs[0:RING]
    cidx = refs[RING:2 * RING]
    bufs = refs[2 * RING:3 * RING]
    acc_sh = refs[3 * RING]
    gsem = refs[3 * RING + 1]
    ssem = refs[3 * RING + 2]
    isem = refs[3 * RING + 3]

    c = lax.axis_index("c")
    s = lax.axis_index("s")
    wid = s * NC + c
    base = pl.multiple_of(s * MSG_RPT, 8)
    count = (N_CHUNKS - wid + NW - 1) // NW

    def fill(j, k):
        off = pl.multiple_of((wid + j * NW) * CHUNK, 8)
        pltpu.async_copy(ei_hbm.at[pl.ds(off, CHUNK)], ridx[k], isem.at[k])
        pltpu.async_copy(ei_hbm.at[pl.ds(N_EDGES + off, CHUNK)], cidx[k],
                         isem.at[k])

    def iwait(k):
        pltpu.make_async_copy(ei_hbm.at[pl.ds(0, CHUNK)], ridx[k],
                              isem.at[k]).wait()
        pltpu.make_async_copy(ei_hbm.at[pl.ds(0, CHUNK)], cidx[k],
                              isem.at[k]).wait()

    def gather(k):
        pltpu.async_copy(g_hbm.at[ridx[k]], bufs[k], gsem.at[k])

    def gwait(k):
        pltpu.make_async_copy(g_hbm.at[pl.ds(0, CHUNK)], bufs[k],
                              gsem.at[k]).wait()

    def scat(k):
        pltpu.async_copy(bufs[k], acc_sh.at[cidx[k]], ssem.at[k], add=True)

    def swait(k):
        pltpu.make_async_copy(bufs[k], acc_sh.at[cidx[k]], ssem.at[k]).wait()

    for k in range(RING):
        fill(k, k)

    zero16 = jnp.zeros((16,), jnp.float32)

    def zrow(r, carry):
        for k8 in range(D // 16):
            bufs[0][r, pl.ds(k8 * 16, 16)] = zero16
        return carry

    lax.fori_loop(0, CHUNK, zrow, 0)
    for t in range(4):
        pltpu.sync_copy(bufs[0], acc_sh.at[pl.ds(base + t * CHUNK, CHUNK)])

    @pl.when(s < NS - 1)
    def _():
        pltpu.sync_copy(bufs[0].at[pl.ds(0, MSG_RPT - 4 * CHUNK)],
                        acc_sh.at[pl.ds(base + 4 * CHUNK,
                                        MSG_RPT - 4 * CHUNK)])

    @pl.when(s == NS - 1)
    def _():
        pltpu.sync_copy(bufs[0].at[pl.ds(0, N_NODES - 15 * MSG_RPT - 4 * CHUNK)],
                        acc_sh.at[pl.ds(base + 4 * CHUNK,
                                        N_NODES - 15 * MSG_RPT - 4 * CHUNK)])

    plsc.subcore_barrier()

    for k in range(RING):
        iwait(k)
        gather(k)

    def block(b, carry):
        jb = b * RING
        for k in range(RING):
            def consume(k=k):
                gwait(k)
                scat(k)
            pl.when(jb + k < count)(consume)
        for k in range(RING):
            jn = jb + RING + k

            def refill(jn=jn, k=k):
                swait(k)
                fill(jn, k)
                iwait(k)
                gather(k)
            pl.when(jn < count)(refill)
        return carry

    lax.fori_loop(0, BLOCKS, block, 0)
    for k in range(RING):
        pl.when(k < count)(functools.partial(swait, k))
    plsc.subcore_barrier()

    @pl.when(s < NS - 1)
    def _():
        sl = pl.ds(base, MSG_RPT)
        pltpu.sync_copy(acc_sh.at[sl], out_hbm.at[c, sl])

    @pl.when(s == NS - 1)
    def _():
        sl = pl.ds(base, N_NODES - 15 * MSG_RPT)
        pltpu.sync_copy(acc_sh.at[sl], out_hbm.at[c, sl])


_msg_call = functools.partial(
    pl.kernel,
    out_type=jax.ShapeDtypeStruct((NC, N_NODES, D), jnp.float32),
    mesh=_sc_mesh,
    scratch_types=(
        [pltpu.VMEM((CHUNK,), jnp.int32)] * (2 * RING)
        + [pltpu.VMEM((CHUNK, D), jnp.float32)] * RING
        + [
            pltpu.VMEM_SHARED((N_NODES, D), jnp.float32),
            pltpu.SemaphoreType.DMA((RING,)),
            pltpu.SemaphoreType.DMA((RING,)),
            pltpu.SemaphoreType.DMA((RING,)),
        ]
    ),
)(_msg_body)


TC_BLK = 2000


def _g_body(x_ref, w_ref, b_ref, degp_ref, g_ref):
    h = lax.dot_general(x_ref[...], w_ref[...],
                        (((1,), (1,)), ((), ())),
                        preferred_element_type=jnp.float32) + b_ref[...]
    d = degp_ref[0, :, :1] + degp_ref[1, :, :1]
    dis = jnp.where(d > 0, lax.rsqrt(d), 0.0)
    g_ref[...] = h * dis


def _g_call(x, W, b2d, degp):
    return pl.pallas_call(
        _g_body,
        grid=(N_NODES // TC_BLK,),
        in_specs=[
            pl.BlockSpec((TC_BLK, D), lambda i: (i, 0)),
            pl.BlockSpec((D, D), lambda i: (0, 0)),
            pl.BlockSpec((1, D), lambda i: (0, 0)),
            pl.BlockSpec((NC, TC_BLK, D), lambda i: (0, i, 0)),
        ],
        out_specs=pl.BlockSpec((TC_BLK, D), lambda i: (i, 0)),
        out_shape=jax.ShapeDtypeStruct((N_NODES, D), jnp.float32),
    )(x, W, b2d, degp)


def _out_body(p_ref, degp_ref, o_ref):
    d = degp_ref[0, :, :1] + degp_ref[1, :, :1]
    dis = jnp.where(d > 0, lax.rsqrt(d), 0.0)
    o_ref[...] = (p_ref[0] + p_ref[1]) * dis


def _out_call(partial, degp):
    return pl.pallas_call(
        _out_body,
        grid=(N_NODES // TC_BLK,),
        in_specs=[
            pl.BlockSpec((NC, TC_BLK, D), lambda i: (0, i, 0)),
            pl.BlockSpec((NC, TC_BLK, D), lambda i: (0, i, 0)),
        ],
        out_specs=pl.BlockSpec((TC_BLK, D), lambda i: (i, 0)),
        out_shape=jax.ShapeDtypeStruct((N_NODES, D), jnp.float32),
    )(partial, degp)


def kernel(x, edge_index, W, b):
    ei1d = edge_index.reshape(2 * N_EDGES)
    degp = _deg_call(ei1d)
    g = _g_call(x, W, b.reshape(1, D), degp)
    partial = _msg_call(ei1d, g)
    return _out_call(partial, degp)


# speedup vs baseline: 1.0162x; 1.0029x over previous
import functools

import jax
import jax.numpy as jnp
from jax import lax
from jax.experimental import pallas as pl
from jax.experimental.pallas import tpu as pltpu
from jax.experimental.pallas import tpu_sc as plsc

N_NODES = 10000
N_EDGES = 320000
D = 128
CHUNK = 128
N_CHUNKS = N_EDGES // CHUNK
NC = 2
NS = 16
NW = NC * NS
N_PAD = 10240
ROWS_PER_TILE = N_PAD // NS
MAX_CH = 79
RING = 3
BLOCKS = (MAX_CH + RING - 1) // RING

_sc_mesh = plsc.VectorSubcoreMesh(core_axis_name="c", subcore_axis_name="s",
                                  num_cores=NC, num_subcores=NS)


def _deg_body(ei_hbm, out_hbm, idx_v, hist_v, red_v, outloc_v, stage_sh, sem):
    c = lax.axis_index("c")
    s = lax.axis_index("s")
    wid = s * NC + c
    zero16 = jnp.zeros((16,), jnp.float32)
    ones16 = jnp.full((16,), 1.0, jnp.float32)

    eoff = pl.multiple_of(N_EDGES + wid * (N_EDGES // NW), 8)
    idx_dma = pltpu.async_copy(ei_hbm.at[pl.ds(eoff, N_EDGES // NW)], idx_v,
                               sem)

    def zbody(i, carry):
        hist_v[pl.ds(pl.multiple_of(i * 16, 16), 16)] = zero16
        return carry

    lax.fori_loop(0, N_PAD // 16, zbody, 0)
    idx_dma.wait()

    def body(m, carry):
        idx16 = idx_v[pl.ds(pl.multiple_of(m * 16, 16), 16)]
        plsc.addupdate_scatter(hist_v, [idx16], ones16)
        return carry

    lax.fori_loop(0, N_EDGES // NW // 16, body, 0)

    pltpu.sync_copy(hist_v, stage_sh.at[s])
    plsc.subcore_barrier()
    rbase = pl.multiple_of(s * ROWS_PER_TILE, 8)
    pltpu.sync_copy(stage_sh.at[:, pl.ds(rbase, ROWS_PER_TILE)], red_v)
    lane0 = jnp.zeros((16,), jnp.int32)

    def rbody(m, carry):
        o = pl.multiple_of(m * 16, 16)
        acc = red_v[0, pl.ds(o, 16)]
        for t in range(1, NS):
            acc = acc + red_v[t, pl.ds(o, 16)]
        rows = m * 16 + lax.iota(jnp.int32, 16)
        plsc.store_scatter(outloc_v, [rows, lane0], acc)
        return carry

    lax.fori_loop(0, ROWS_PER_TILE // 16, rbody, 0)
    pltpu.sync_copy(outloc_v, out_hbm.at[c, pl.ds(rbase, ROWS_PER_TILE)])


_deg_call = functools.partial(
    pl.kernel,
    out_type=jax.ShapeDtypeStruct((NC, N_PAD, D), jnp.float32),
    mesh=_sc_mesh,
    scratch_types=[
        pltpu.VMEM((N_EDGES // NW,), jnp.int32),
        pltpu.VMEM((N_PAD,), jnp.float32),
        pltpu.VMEM((NS, ROWS_PER_TILE), jnp.float32),
        pltpu.VMEM((ROWS_PER_TILE, D), jnp.float32),
        pltpu.VMEM_SHARED((NS, N_PAD), jnp.float32),
        pltpu.SemaphoreType.DMA,
    ],
    compiler_params=pltpu.CompilerParams(needs_layout_passes=False,
                                         use_tc_tiling_on_sc=False),
)(_deg_body)


MSG_RPT = 632


def _msg_body(ei_hbm, g_hbm, out_hbm, *refs):
    ridx = refs[0:RING]
    cidx = refs[RING:2 * RING]
    bufs = refs[2 * RING:3 * RING]
    acc_sh = refs[3 * RING]
    gsem = refs[3 * RING + 1]
    ssem = refs[3 * RING + 2]
    isem = refs[3 * RING + 3]

    c = lax.axis_index("c")
    s = lax.axis_index("s")
    wid = s * NC + c
    base = pl.multiple_of(s * MSG_RPT, 8)
    count = (N_CHUNKS - wid + NW - 1) // NW

    def fill_r(j, k):
        off = pl.multiple_of((wid + j * NW) * CHUNK, 8)
        pltpu.async_copy(ei_hbm.at[pl.ds(off, CHUNK)], ridx[k], isem.at[k])

    def fill_c(j, k):
        off = pl.multiple_of((wid + j * NW) * CHUNK, 8)
        pltpu.async_copy(ei_hbm.at[pl.ds(N_EDGES + off, CHUNK)], cidx[k],
                         isem.at[k])

    def fill(j, k):
        fill_r(j, k)
        fill_c(j, k)

    def iwait(k):
        pltpu.make_async_copy(ei_hbm.at[pl.ds(0, CHUNK)], ridx[k],
                              isem.at[k]).wait()
        pltpu.make_async_copy(ei_hbm.at[pl.ds(0, CHUNK)], cidx[k],
                              isem.at[k]).wait()

    def gather(k):
        pltpu.async_copy(g_hbm.at[ridx[k]], bufs[k], gsem.at[k])

    def gwait(k):
        pltpu.make_async_copy(g_hbm.at[pl.ds(0, CHUNK)], bufs[k],
                              gsem.at[k]).wait()

    def scat(k):
        pltpu.async_copy(bufs[k], acc_sh.at[cidx[k]], ssem.at[k], add=True)

    def swait(k):
        pltpu.make_async_copy(bufs[k], acc_sh.at[cidx[k]], ssem.at[k]).wait()

    for k in range(RING):
        fill(k, k)

    zero16 = jnp.zeros((16,), jnp.float32)

    def zrow(r, carry):
        for k8 in range(D // 16):
            bufs[0][r, pl.ds(k8 * 16, 16)] = zero16
        return carry

    lax.fori_loop(0, CHUNK, zrow, 0)
    for t in range(4):
        pltpu.sync_copy(bufs[0], acc_sh.at[pl.ds(base + t * CHUNK, CHUNK)])

    @pl.when(s < NS - 1)
    def _():
        pltpu.sync_copy(bufs[0].at[pl.ds(0, MSG_RPT - 4 * CHUNK)],
                        acc_sh.at[pl.ds(base + 4 * CHUNK,
                                        MSG_RPT - 4 * CHUNK)])

    @pl.when(s == NS - 1)
    def _():
        pltpu.sync_copy(bufs[0].at[pl.ds(0, N_NODES - 15 * MSG_RPT - 4 * CHUNK)],
                        acc_sh.at[pl.ds(base + 4 * CHUNK,
                                        N_NODES - 15 * MSG_RPT - 4 * CHUNK)])

    plsc.subcore_barrier()

    for k in range(RING):
        iwait(k)
        gather(k)

    def block(b, carry):
        jb = b * RING
        for k in range(RING):
            def consume(k=k):
                gwait(k)
                scat(k)
            pl.when(jb + k < count)(consume)
        for k in range(RING):
            jn = jb + RING + k

            def refill(jn=jn, k=k):
                fill_r(jn, k)
                swait(k)
                fill_c(jn, k)
                iwait(k)
                gather(k)
            pl.when(jn < count)(refill)
        return carry

    lax.fori_loop(0, BLOCKS, block, 0)
    for k in range(RING):
        pl.when(k < count)(functools.partial(swait, k))
    plsc.subcore_barrier()

    @pl.when(s < NS - 1)
    def _():
        sl = pl.ds(base, MSG_RPT)
        pltpu.sync_copy(acc_sh.at[sl], out_hbm.at[c, sl])

    @pl.when(s == NS - 1)
    def _():
        sl = pl.ds(base, N_NODES - 15 * MSG_RPT)
        pltpu.sync_copy(acc_sh.at[sl], out_hbm.at[c, sl])


_msg_call = functools.partial(
    pl.kernel,
    out_type=jax.ShapeDtypeStruct((NC, N_NODES, D), jnp.float32),
    mesh=_sc_mesh,
    scratch_types=(
        [pltpu.VMEM((CHUNK,), jnp.int32)] * (2 * RING)
        + [pltpu.VMEM((CHUNK, D), jnp.float32)] * RING
        + [
            pltpu.VMEM_SHARED((N_NODES, D), jnp.float32),
            pltpu.SemaphoreType.DMA((RING,)),
            pltpu.SemaphoreType.DMA((RING,)),
            pltpu.SemaphoreType.DMA((RING,)),
        ]
    ),
)(_msg_body)


TC_BLK = 2000


def _g_body(x_ref, w_ref, b_ref, degp_ref, g_ref):
    h = lax.dot_general(x_ref[...], w_ref[...],
                        (((1,), (1,)), ((), ())),
                        preferred_element_type=jnp.float32) + b_ref[...]
    d = degp_ref[0, :, :1] + degp_ref[1, :, :1]
    dis = jnp.where(d > 0, lax.rsqrt(d), 0.0)
    g_ref[...] = h * dis


def _g_call(x, W, b2d, degp):
    return pl.pallas_call(
        _g_body,
        grid=(N_NODES // TC_BLK,),
        in_specs=[
            pl.BlockSpec((TC_BLK, D), lambda i: (i, 0)),
            pl.BlockSpec((D, D), lambda i: (0, 0)),
            pl.BlockSpec((1, D), lambda i: (0, 0)),
            pl.BlockSpec((NC, TC_BLK, D), lambda i: (0, i, 0)),
        ],
        out_specs=pl.BlockSpec((TC_BLK, D), lambda i: (i, 0)),
        out_shape=jax.ShapeDtypeStruct((N_NODES, D), jnp.float32),
    )(x, W, b2d, degp)


def _out_body(p_ref, degp_ref, o_ref):
    d = degp_ref[0, :, :1] + degp_ref[1, :, :1]
    dis = jnp.where(d > 0, lax.rsqrt(d), 0.0)
    o_ref[...] = (p_ref[0] + p_ref[1]) * dis


def _out_call(partial, degp):
    return pl.pallas_call(
        _out_body,
        grid=(N_NODES // TC_BLK,),
        in_specs=[
            pl.BlockSpec((NC, TC_BLK, D), lambda i: (0, i, 0)),
            pl.BlockSpec((NC, TC_BLK, D), lambda i: (0, i, 0)),
        ],
        out_specs=pl.BlockSpec((TC_BLK, D), lambda i: (i, 0)),
        out_shape=jax.ShapeDtypeStruct((N_NODES, D), jnp.float32),
    )(partial, degp)


def kernel(x, edge_index, W, b):
    ei1d = edge_index.reshape(2 * N_EDGES)
    degp = _deg_call(ei1d)
    g = _g_call(x, W, b.reshape(1, D), degp)
    partial = _msg_call(ei1d, g)
    return _out_call(partial, degp)
